# baseline (speedup 1.0000x reference)
import functools

import jax
import jax.numpy as jnp
from jax import lax
from jax.experimental import pallas as pl
from jax.experimental.pallas import tpu as pltpu
from jax.experimental.pallas import tpu_sc as plsc

NUM_CODES = 1024
CODE_DIM = 64

TM = 1024
NC, NS = 2, 16
NW = NC * NS
TPW = 1024 // NW


def _tc_body(z_ref, cb_ref, idx_ref):
    z = z_ref[...]
    cb = cb_ref[...]
    K = NUM_CODES
    T = z.shape[0]
    cb_hi = cb.astype(jnp.bfloat16)
    r_hi = cb - cb_hi.astype(jnp.float32)
    cb_mid = r_hi.astype(jnp.bfloat16)
    cb_lo = (r_hi - cb_mid.astype(jnp.float32)).astype(jnp.bfloat16)
    cn = jnp.sum(cb * cb, axis=1, keepdims=True)
    cn_hi = cn.astype(jnp.bfloat16)
    cn_mid = (cn - cn_hi.astype(jnp.float32)).astype(jnp.bfloat16)
    cn_lo = (cn - cn_hi.astype(jnp.float32)
             - cn_mid.astype(jnp.float32)).astype(jnp.bfloat16)
    neg2 = jnp.bfloat16(-2.0)
    cb4 = jnp.concatenate([neg2 * cb_hi, neg2 * cb_hi, neg2 * cb_mid,
                           cn_hi, cn_mid, cn_lo], axis=1)
    z_hi = z.astype(jnp.bfloat16)
    z_lo = (z - z_hi.astype(jnp.float32)).astype(jnp.bfloat16)
    ones = jnp.ones((T, 3), jnp.bfloat16)
    z4 = jnp.concatenate([z_hi, z_lo, z_hi, ones], axis=1)
    S = lax.dot_general(z4, cb4, (((1,), (1,)), ((), ())),
                        preferred_element_type=jnp.float32)
    iota = lax.broadcasted_iota(jnp.int32, (T, K), 1)
    m1 = jnp.min(S, axis=1, keepdims=True)
    i1 = jnp.min(jnp.where(S == m1, iota, K), axis=1, keepdims=True)
    S2 = jnp.where(iota == i1, jnp.inf, S)
    m2 = jnp.min(S2, axis=1, keepdims=True)
    i2 = jnp.min(jnp.where(S2 == m2, iota, K), axis=1, keepdims=True)
    oh1 = (iota == i1).astype(jnp.bfloat16)
    oh2 = (iota == i2).astype(jnp.bfloat16)
    dn = (((1,), (0,)), ((), ()))

    def orow(oh):
        return (lax.dot_general(oh, cb_hi, dn, preferred_element_type=jnp.float32)
                + lax.dot_general(oh, cb_mid, dn, preferred_element_type=jnp.float32)
                + lax.dot_general(oh, cb_lo, dn, preferred_element_type=jnp.float32))

    r1 = orow(oh1)
    r2 = orow(oh2)
    d1 = jnp.sqrt(jnp.sum((z - r1) ** 2, axis=1, keepdims=True))
    d2 = jnp.sqrt(jnp.sum((z - r2) ** 2, axis=1, keepdims=True))
    take2 = (d2 < d1) | ((d2 == d1) & (i2 < i1))
    idx_ref[...] = jnp.where(take2, i2, i1)


def _sc_body(cb_hbm, idx_hbm, zq_out, idx_v, zq_v, sem):
    wid = lax.axis_index("s") * NC + lax.axis_index("c")
    base = wid * TPW
    pltpu.async_copy(idx_hbm.at[pl.ds(base, TPW)], idx_v, sem).wait()
    pltpu.async_copy(cb_hbm.at[idx_v], zq_v, sem).wait()
    pltpu.sync_copy(zq_v, zq_out.at[pl.ds(base, TPW)])


def kernel(z_e, codebook):
    B, S_len, D = z_e.shape
    T = B * S_len
    z2 = z_e.reshape(T, D)
    idx = pl.pallas_call(
        _tc_body,
        grid=(T // TM,),
        in_specs=[
            pl.BlockSpec((TM, D), lambda i: (i, 0)),
            pl.BlockSpec((NUM_CODES, D), lambda i: (0, 0)),
        ],
        out_specs=pl.BlockSpec((TM, 1), lambda i: (i, 0)),
        out_shape=jax.ShapeDtypeStruct((T, 1), jnp.int32),
    )(z2, codebook)

    sc_gather = functools.partial(
        pl.kernel,
        out_type=jax.ShapeDtypeStruct((T, D), jnp.float32),
        mesh=plsc.VectorSubcoreMesh(core_axis_name="c", subcore_axis_name="s",
                                    num_cores=NC, num_subcores=NS),
        compiler_params=pltpu.CompilerParams(needs_layout_passes=False,
                                             use_tc_tiling_on_sc=False),
        scratch_types=[
            pltpu.VMEM((TPW,), jnp.int32),
            pltpu.VMEM((TPW, D), jnp.float32),
            pltpu.SemaphoreType.DMA,
        ],
    )(_sc_body)
    zq = sc_gather(codebook, idx.reshape(T))
    return (zq.reshape(B, S_len, D), idx.reshape(B, S_len))
